# baseline (speedup 1.0000x reference)
import functools

import jax
import jax.numpy as jnp
from jax import lax
from jax.experimental import pallas as pl
from jax.experimental.pallas import tpu as pltpu
from jax.experimental.pallas import tpu_sc as plsc

F32 = jnp.float32
I32 = jnp.int32

_H = 8
_C = 128
_NW = 32
_B1 = 80
_B2 = 16


def _tc_stage1(x, Wn, bn2, Wg, Vc):
    n, d = x.shape
    bn_rows = 2000
    grid = (n // bn_rows,)
    hw = Wg.shape[1]

    def body(x_ref, wn_ref, bn_ref, wg_ref, vc_ref, h_ref, h16_ref,
             cb_ref):
        h0 = jnp.dot(x_ref[...], wn_ref[...],
                     preferred_element_type=F32) + bn_ref[...]
        hv = jnp.dot(h0, wg_ref[...], preferred_element_type=F32)
        h_ref[...] = hv
        h16_ref[...] = hv.astype(jnp.bfloat16)
        cb_ref[...] = jnp.dot(h0, vc_ref[...], preferred_element_type=F32)

    return pl.pallas_call(
        body,
        grid=grid,
        in_specs=[
            pl.BlockSpec((bn_rows, d), lambda i: (i, 0)),
            pl.BlockSpec((d, d), lambda i: (0, 0)),
            pl.BlockSpec((1, d), lambda i: (0, 0)),
            pl.BlockSpec((d, hw), lambda i: (0, 0)),
            pl.BlockSpec((d, _C), lambda i: (0, 0)),
        ],
        out_specs=[
            pl.BlockSpec((bn_rows, hw), lambda i: (i, 0)),
            pl.BlockSpec((bn_rows, hw), lambda i: (i, 0)),
            pl.BlockSpec((bn_rows, _C), lambda i: (i, 0)),
        ],
        out_shape=[
            jax.ShapeDtypeStruct((n, hw), F32),
            jax.ShapeDtypeStruct((n, hw), jnp.bfloat16),
            jax.ShapeDtypeStruct((n, _C), F32),
        ],
    )(x, Wn, bn2, Wg, Vc)


def _tc_stage2(combo, dpart):
    n = combo.shape[0]
    bn_rows = 2000
    grid = (n // bn_rows,)

    def body(cb_ref, dp_ref, rd_ref, wl_ref):
        cb = cb_ref[...]
        al = cb[:, 0:16] + cb[:, 16:32]
        exl = jnp.exp(jnp.maximum(al, 0.2 * al))
        den = dp_ref[0][:, 0:16] + dp_ref[1][:, 0:16] + exl
        rd = 1.0 / den
        rd_ref[...] = jnp.concatenate(
            [rd, jnp.zeros((rd.shape[0], _C - 16), F32)], axis=1)
        wl_ref[...] = exl * rd

    return pl.pallas_call(
        body,
        grid=grid,
        in_specs=[
            pl.BlockSpec((bn_rows, _C), lambda i: (i, 0)),
            pl.BlockSpec((2, bn_rows, _C), lambda i: (0, i, 0)),
        ],
        out_specs=[
            pl.BlockSpec((bn_rows, _C), lambda i: (i, 0)),
            pl.BlockSpec((bn_rows, 16), lambda i: (i, 0)),
        ],
        out_shape=[
            jax.ShapeDtypeStruct((n, _C), F32),
            jax.ShapeDtypeStruct((n, 16), F32),
        ],
    )(combo, dpart)


def _tc_stage3(opart, h, wl_dup, bg2, Punp):
    n = h.shape[0]
    bn_rows = 2000
    grid = (n // bn_rows,)
    hw = h.shape[1]

    def body(op_ref, h_ref, wl_ref, bg_ref, p_ref, o_ref):
        acc = jnp.dot(op_ref[0] + op_ref[1], p_ref[...],
                      preferred_element_type=F32)
        wl = wl_ref[...]
        hv = h_ref[...]
        for hh in range(_H):
            acc = acc + wl[:, hh:hh + 1] * hv[:, hh * _C:(hh + 1) * _C]
        o_ref[...] = jnp.tanh(acc * (1.0 / _H) + bg_ref[...])

    return pl.pallas_call(
        body,
        grid=grid,
        in_specs=[
            pl.BlockSpec((2, bn_rows, _C), lambda i: (0, i, 0)),
            pl.BlockSpec((bn_rows, hw), lambda i: (i, 0)),
            pl.BlockSpec((bn_rows, 16), lambda i: (i, 0)),
            pl.BlockSpec((1, _C), lambda i: (0, 0)),
            pl.BlockSpec((_C, _C), lambda i: (0, 0)),
        ],
        out_specs=pl.BlockSpec((bn_rows, _C), lambda i: (i, 0)),
        out_shape=jax.ShapeDtypeStruct((n, _C), F32),
    )(opart, h, wl_dup, bg2, Punp)


def _tc_rel(rel_embed, Wr, br2):
    r, d = rel_embed.shape

    def body(re_ref, wr_ref, br_ref, o_ref):
        o_ref[...] = jnp.dot(re_ref[...], wr_ref[...],
                             preferred_element_type=F32) + br_ref[...]

    return pl.pallas_call(
        body,
        out_shape=jax.ShapeDtypeStruct((r, d), F32),
    )(rel_embed, Wr, br2)


def _sc_pass1(src, dst, combo, n):
    e = src.shape[0]
    ew = e // _NW
    nr = n // 8
    nrp = ((nr + 127) // 128) * 128
    rstride = nrp // 16
    nch = ew // _B1
    mesh = plsc.VectorSubcoreMesh(core_axis_name="c", subcore_axis_name="s")

    @functools.partial(
        pl.kernel,
        out_type=(
            jax.ShapeDtypeStruct((e, 16), F32),
            jax.ShapeDtypeStruct((2, nrp, _C), F32),
        ),
        mesh=mesh,
        scratch_types=[
            pltpu.VMEM((2, _B1), I32),
            pltpu.VMEM((2, _B1), I32),
            pltpu.VMEM((2, _B1), I32),
            pltpu.VMEM((2, _B1), I32),
            pltpu.VMEM((2, _B1, _C), F32),
            pltpu.VMEM((2, _B1, _C), F32),
            pltpu.VMEM((2, _B1, 16), F32),
            pltpu.VMEM((2, _B1, _C), F32),
            pltpu.VMEM((16, _C), F32),
            pltpu.VMEM_SHARED((nrp, _C), F32),
            pltpu.SemaphoreType.DMA,
            pltpu.SemaphoreType.DMA,
            pltpu.SemaphoreType.DMA,
            pltpu.SemaphoreType.DMA,
            pltpu.SemaphoreType.DMA,
        ],
    )
    def kern(src_hbm, dst_hbm, cb_hbm, ex_hbm, dp_hbm,
             srcv, dstv, rowv, colv, sbuf, dbuf, exbuf, pay, zbuf, dacc,
             semg0, semg1, semo0, semo1, semi):
        cid = lax.axis_index("c")
        sid = lax.axis_index("s")
        wid = sid * 2 + cid
        base = wid * ew
        semg = (semg0, semg1)
        semo = (semo0, semo1)

        zv = jnp.zeros((16,), F32)
        ziv = jnp.zeros((16,), I32)

        def zrow(i, carry):
            for k in range(_C // 16):
                zbuf[i, pl.ds(k * 16, 16)] = zv
            return carry

        lax.fori_loop(0, 16, zrow, 0)
        for q in range(rstride // 16):
            pltpu.sync_copy(zbuf, dacc.at[pl.ds(sid * rstride + q * 16, 16)])

        def zpay(i, carry):
            for s in range(2):
                for k in range(_C // 16):
                    pay[s, i, pl.ds(k * 16, 16)] = zv
            return carry

        lax.fori_loop(0, _B1, zpay, 0)
        for s in range(2):
            for g in range(_B1 // 16):
                colv[s, pl.ds(g * 16, 16)] = ziv
        plsc.subcore_barrier()

        def issue(c, s):
            cc = jnp.minimum(c, nch - 1)
            off = base + cc * _B1
            pltpu.async_copy(src_hbm.at[pl.ds(off, _B1)],
                             srcv.at[s], semi).wait()
            pltpu.async_copy(dst_hbm.at[pl.ds(off, _B1)],
                             dstv.at[s], semi).wait()
            pltpu.async_copy(cb_hbm.at[srcv.at[s]], sbuf.at[s], semg[s])
            pltpu.async_copy(cb_hbm.at[dstv.at[s]], dbuf.at[s], semg[s])

        issue(0, 0)
        issue(1, 1)

        def compute(c, s):
            for g in range(_B1 // 16):
                dv = dstv[s, pl.ds(g * 16, 16)]
                rowv[s, pl.ds(g * 16, 16)] = lax.shift_right_logical(dv, 3)
                cv = lax.shift_left(jnp.bitwise_and(dv, 7), 4)
                colv[s, pl.ds(g * 16, 16)] = cv
                for l in range(16):
                    j = g * 16 + l
                    a = sbuf[s, j, pl.ds(0, 16)] + dbuf[s, j, pl.ds(16, 16)]
                    a = jnp.maximum(a, 0.2 * a)
                    ex = jnp.exp(a)
                    exbuf[s, j, :] = ex
                    pay[s, j, pl.ds(cv[l], 16)] = ex
            off = base + c * _B1
            pltpu.sync_copy(exbuf.at[s], ex_hbm.at[pl.ds(off, _B1)])
            pltpu.async_copy(pay.at[s], dacc.at[rowv.at[s]], semo[s],
                             add=True)
            issue(c + 2, s)

        def wait_gathers(s):
            pltpu.make_async_copy(
                cb_hbm.at[srcv.at[s]], sbuf.at[s], semg[s]).wait()
            pltpu.make_async_copy(
                cb_hbm.at[dstv.at[s]], dbuf.at[s], semg[s]).wait()

        def wait_outputs(s):
            pltpu.make_async_copy(
                pay.at[s], dacc.at[rowv.at[s]], semo[s]).wait()

        def zero_windows(s):
            for g in range(_B1 // 16):
                cvz = colv[s, pl.ds(g * 16, 16)]
                for l in range(16):
                    pay[s, g * 16 + l, pl.ds(cvz[l], 16)] = zv

        wait_gathers(0)
        compute(0, 0)
        wait_gathers(1)
        compute(1, 1)

        def slotstep(c, s):
            wait_gathers(s)
            wait_outputs(s)
            zero_windows(s)
            compute(c, s)

        def outer(i, carry):
            slotstep(2 * i, 0)
            slotstep(2 * i + 1, 1)
            return carry

        lax.fori_loop(1, nch // 2, outer, 0)
        slotstep(nch - 1, 0)
        for s in range(2):
            wait_gathers(s)
            wait_outputs(s)
        plsc.subcore_barrier()
        pltpu.sync_copy(dacc.at[pl.ds(sid * rstride, rstride)],
                        dp_hbm.at[cid, pl.ds(sid * rstride, rstride)])

    return kern(src, dst, combo)


def _sc_pass2(src, dst, ex_all, rd_pad, h, n):
    e = src.shape[0]
    b2 = _B2
    ew = e // _NW
    nch = ew // b2
    rstride = ((n // 16) // 8) * 8
    rsize = n - 15 * rstride
    hw = 2 * h.shape[1]
    mesh = plsc.VectorSubcoreMesh(core_axis_name="c", subcore_axis_name="s")

    @functools.partial(
        pl.kernel,
        out_type=jax.ShapeDtypeStruct((2, n, _C), F32),
        mesh=mesh,
        scratch_types=[
            pltpu.VMEM((2, b2), I32),
            pltpu.VMEM((2, b2), I32),
            pltpu.VMEM((2, b2), I32),
            pltpu.VMEM((b2, 16), F32),
            pltpu.VMEM((b2, 16), F32),
            pltpu.VMEM((b2, _C), F32),
            pltpu.VMEM((b2, _C), F32),
            pltpu.VMEM((b2, hw // 2), I32),
            pltpu.VMEM((b2, hw // 2), I32),
            pltpu.VMEM((b2, _C), F32),
            pltpu.VMEM((b2, _C), F32),
            pltpu.VMEM((16, _C), F32),
            pltpu.VMEM_SHARED((n, _C), F32),
            pltpu.SemaphoreType.DMA,
            pltpu.SemaphoreType.DMA,
            pltpu.SemaphoreType.DMA,
            pltpu.SemaphoreType.DMA,
            pltpu.SemaphoreType.DMA,
        ],
    )
    def kern(src_hbm, dst_hbm, ex_hbm, rd_hbm, h_hbm, out_hbm,
             srcvs, dstvs, dstw, exv0, exv1, rdv0, rdv1, hbuf0, hbuf1,
             mbuf0, mbuf1, zbuf, oacc,
             semg0, semg1, sems0, sems1, semi):
        cid = lax.axis_index("c")
        sid = lax.axis_index("s")
        wid = sid * 2 + cid
        base = wid * ew
        slots = (
            (exv0, rdv0, hbuf0, mbuf0, semg0, sems0),
            (exv1, rdv1, hbuf1, mbuf1, semg1, sems1),
        )

        zv = jnp.zeros((16,), F32)

        def zrow(i, carry):
            for k in range(_C // 16):
                zbuf[i, pl.ds(k * 16, 16)] = zv
            return carry

        lax.fori_loop(0, 16, zrow, 0)
        for q in range(rsize // 16):
            pltpu.sync_copy(zbuf, oacc.at[pl.ds(sid * rstride + q * 16, 16)])

        def zmb(i, carry):
            for k in range(_C // 16):
                mbuf0[i, pl.ds(k * 16, 16)] = zv
                mbuf1[i, pl.ds(k * 16, 16)] = zv
            return carry

        ziv = jnp.zeros((16,), I32)
        for s in range(2):
            for g in range(b2 // 16):
                srcvs[s, pl.ds(g * 16, 16)] = ziv
                dstvs[s, pl.ds(g * 16, 16)] = ziv
                dstw[s, pl.ds(g * 16, 16)] = ziv
        lax.fori_loop(0, b2, zmb, 0)
        plsc.subcore_barrier()

        def issue(c, s):
            exv, rdv, hbuf, _, semg, _ = slots[s]
            cc = jnp.minimum(c, nch - 1)
            off = base + cc * b2
            pltpu.async_copy(
                src_hbm.at[pl.ds(off, b2)], srcvs.at[s], semi).wait()
            pltpu.async_copy(
                dst_hbm.at[pl.ds(off, b2)], dstvs.at[s], semi).wait()
            pltpu.async_copy(ex_hbm.at[pl.ds(off, b2)], exv, semg)
            pltpu.async_copy(rd_hbm.at[dstvs.at[s]], rdv, semg)
            pltpu.async_copy(h_hbm.at[srcvs.at[s]], hbuf, semg)

        pltpu.async_copy(mbuf0, oacc.at[dstw.at[0]], sems0, add=True)
        pltpu.async_copy(mbuf1, oacc.at[dstw.at[1]], sems1, add=True)
        issue(0, 0)
        issue(1, 1)

        def slotstep(c, s):
            exv, rdv, hbuf, mbuf, semg, sems = slots[s]
            pltpu.make_async_copy(
                ex_hbm.at[pl.ds(base, b2)], exv, semg).wait()
            pltpu.make_async_copy(rd_hbm.at[dstvs.at[s]], rdv, semg).wait()
            pltpu.make_async_copy(h_hbm.at[srcvs.at[s]], hbuf, semg).wait()
            pltpu.make_async_copy(mbuf, oacc.at[dstw.at[s]], sems).wait()
            for g in range(b2 // 16):
                dstw[s, pl.ds(g * 16, 16)] = dstvs[s, pl.ds(g * 16, 16)]

            def edge(j, carry):
                wv = exv[j] * rdv[j, pl.ds(0, 16)]
                acc = [zv] * (_C // 16)
                for hh in range(_H):
                    sc = wv[hh]
                    for g in range(_C // 32):
                        hv32 = hbuf[j, pl.ds(hh * 64 + g * 16, 16)]
                        ha = lax.bitcast_convert_type(
                            lax.shift_left(hv32, 16), F32)
                        hb = lax.bitcast_convert_type(
                            jnp.bitwise_and(hv32, jnp.int32(-65536)), F32)
                        acc[2 * g] = acc[2 * g] + sc * ha
                        acc[2 * g + 1] = acc[2 * g + 1] + sc * hb
                for k in range(_C // 16):
                    mbuf[j, pl.ds(k * 16, 16)] = acc[k]
                return carry

            lax.fori_loop(0, b2, edge, 0, unroll=2)
            pltpu.async_copy(mbuf, oacc.at[dstw.at[s]], sems, add=True)
            issue(c + 2, s)

        def outer(i, carry):
            slotstep(2 * i, 0)
            slotstep(2 * i + 1, 1)
            return carry

        lax.fori_loop(0, nch // 2, outer, 0)
        if nch % 2 == 1:
            slotstep(nch - 1, 0)
        for s in range(2):
            exv, rdv, hbuf, mbuf, semg, sems = slots[s]
            pltpu.make_async_copy(
                ex_hbm.at[pl.ds(base, b2)], exv, semg).wait()
            pltpu.make_async_copy(rd_hbm.at[dstvs.at[s]], rdv, semg).wait()
            pltpu.make_async_copy(h_hbm.at[srcvs.at[s]], hbuf, semg).wait()
            pltpu.make_async_copy(mbuf, oacc.at[dstw.at[s]], sems).wait()
        plsc.subcore_barrier()
        pltpu.sync_copy(oacc.at[pl.ds(sid * rstride, rsize)],
                        out_hbm.at[cid, pl.ds(sid * rstride, rsize)])

    return kern(src, dst, ex_all, rd_pad, h)


def kernel(x, edge_index, edge_attr, edge_type, rel_embed, num_nodes,
           Wn, bn, We, be, Wg, att_src, att_dst, bg, Wr, br):
    n, d = x.shape

    Wg3 = Wg.reshape(d, _H, _C)
    Vs = jnp.einsum('dhk,hk->dh', Wg3, att_src)
    Vd = jnp.einsum('dhk,hk->dh', Wg3, att_dst)
    Vc = jnp.concatenate(
        [Vs, Vs, Vd, Vd, jnp.zeros((d, _C - 32), F32)], axis=1)
    bn2 = bn.reshape(1, -1)
    bg2 = bg.reshape(1, -1)
    br2 = br.reshape(1, -1)

    src = edge_index[0]
    dst = edge_index[1]

    h, h16, combo = _tc_stage1(x, Wn, bn2, Wg, Vc)
    perm = jnp.arange(_C).reshape(-1, 2, 16)
    perm = jnp.transpose(perm, (0, 2, 1)).reshape(_C)
    Punp = jnp.zeros((_C, _C), F32).at[perm, jnp.arange(_C)].set(1.0)
    ex_all, dpart = _sc_pass1(src, dst, combo, n)
    dpart = dpart[:, :n // 8, :].reshape(2, n, 16)
    rd_pad, wl_dup = _tc_stage2(combo, dpart)
    h32 = lax.bitcast_convert_type(
        h16.reshape(n, -1, 2), jnp.int32)
    opart = _sc_pass2(src, dst, ex_all, rd_pad, h32, n)
    x1 = _tc_stage3(opart, h, wl_dup, bg2, Punp)
    rel_out = _tc_rel(rel_embed, Wr, br2)
    return (x1, rel_out)


# speedup vs baseline: 1.4961x; 1.4961x over previous
import functools

import jax
import jax.numpy as jnp
from jax import lax
from jax.experimental import pallas as pl
from jax.experimental.pallas import tpu as pltpu
from jax.experimental.pallas import tpu_sc as plsc

F32 = jnp.float32
I32 = jnp.int32

_H = 8
_C = 128
_NW = 32
_B1 = 80
_B2 = 16


def _tc_stage1(x, Wn, bn2, Wg, Vc):
    n, d = x.shape
    bn_rows = 2000
    grid = (n // bn_rows,)
    hw = Wg.shape[1]

    def body(x_ref, wn_ref, bn_ref, wg_ref, vc_ref, h_ref, cb_ref):
        h0 = jnp.dot(x_ref[...], wn_ref[...],
                     preferred_element_type=F32) + bn_ref[...]
        h_ref[...] = jnp.dot(h0, wg_ref[...], preferred_element_type=F32)
        cb_ref[...] = jnp.dot(h0, vc_ref[...], preferred_element_type=F32)

    return pl.pallas_call(
        body,
        grid=grid,
        in_specs=[
            pl.BlockSpec((bn_rows, d), lambda i: (i, 0)),
            pl.BlockSpec((d, d), lambda i: (0, 0)),
            pl.BlockSpec((1, d), lambda i: (0, 0)),
            pl.BlockSpec((d, hw), lambda i: (0, 0)),
            pl.BlockSpec((d, _C), lambda i: (0, 0)),
        ],
        out_specs=[
            pl.BlockSpec((bn_rows, hw), lambda i: (i, 0)),
            pl.BlockSpec((bn_rows, _C), lambda i: (i, 0)),
        ],
        out_shape=[
            jax.ShapeDtypeStruct((n, hw), F32),
            jax.ShapeDtypeStruct((n, _C), F32),
        ],
    )(x, Wn, bn2, Wg, Vc)


def _tc_stage2(combo, dpart):
    n = combo.shape[0]
    bn_rows = 2000
    grid = (n // bn_rows,)

    def body(cb_ref, dp_ref, rd_ref, wl_ref):
        cb = cb_ref[...]
        al = cb[:, 0:16] + cb[:, 16:32]
        exl = jnp.exp(jnp.maximum(al, 0.2 * al))
        den = dp_ref[0][:, 0:16] + dp_ref[1][:, 0:16] + exl
        rd = 1.0 / den
        rd_ref[...] = jnp.concatenate(
            [rd, jnp.zeros((rd.shape[0], _C - 16), F32)], axis=1)
        wl_ref[...] = exl * rd

    return pl.pallas_call(
        body,
        grid=grid,
        in_specs=[
            pl.BlockSpec((bn_rows, _C), lambda i: (i, 0)),
            pl.BlockSpec((2, bn_rows, _C), lambda i: (0, i, 0)),
        ],
        out_specs=[
            pl.BlockSpec((bn_rows, _C), lambda i: (i, 0)),
            pl.BlockSpec((bn_rows, 16), lambda i: (i, 0)),
        ],
        out_shape=[
            jax.ShapeDtypeStruct((n, _C), F32),
            jax.ShapeDtypeStruct((n, 16), F32),
        ],
    )(combo, dpart)


def _tc_stage3(opart, h, wl_dup, bg2):
    n = h.shape[0]
    bn_rows = 2000
    grid = (n // bn_rows,)
    hw = h.shape[1]

    def body(op_ref, h_ref, wl_ref, bg_ref, o_ref):
        acc = op_ref[0] + op_ref[1]
        wl = wl_ref[...]
        hv = h_ref[...]
        for hh in range(_H):
            acc = acc + wl[:, hh:hh + 1] * hv[:, hh * _C:(hh + 1) * _C]
        o_ref[...] = jnp.tanh(acc * (1.0 / _H) + bg_ref[...])

    return pl.pallas_call(
        body,
        grid=grid,
        in_specs=[
            pl.BlockSpec((2, bn_rows, _C), lambda i: (0, i, 0)),
            pl.BlockSpec((bn_rows, hw), lambda i: (i, 0)),
            pl.BlockSpec((bn_rows, 16), lambda i: (i, 0)),
            pl.BlockSpec((1, _C), lambda i: (0, 0)),
        ],
        out_specs=pl.BlockSpec((bn_rows, _C), lambda i: (i, 0)),
        out_shape=jax.ShapeDtypeStruct((n, _C), F32),
    )(opart, h, wl_dup, bg2)


def _tc_rel(rel_embed, Wr, br2):
    r, d = rel_embed.shape

    def body(re_ref, wr_ref, br_ref, o_ref):
        o_ref[...] = jnp.dot(re_ref[...], wr_ref[...],
                             preferred_element_type=F32) + br_ref[...]

    return pl.pallas_call(
        body,
        out_shape=jax.ShapeDtypeStruct((r, d), F32),
    )(rel_embed, Wr, br2)


def _sc_pass1(src, dst, combo, n):
    e = src.shape[0]
    ew = e // _NW
    nr = n // 8
    nrp = ((nr + 127) // 128) * 128
    rstride = nrp // 16
    nch = ew // _B1
    mesh = plsc.VectorSubcoreMesh(core_axis_name="c", subcore_axis_name="s")

    @functools.partial(
        pl.kernel,
        out_type=(
            jax.ShapeDtypeStruct((e, 16), F32),
            jax.ShapeDtypeStruct((2, nrp, _C), F32),
        ),
        mesh=mesh,
        scratch_types=[
            pltpu.VMEM((ew,), I32),
            pltpu.VMEM((ew,), I32),
            pltpu.VMEM((2, _B1), I32),
            pltpu.VMEM((2, _B1), I32),
            pltpu.VMEM((2, _B1, _C), F32),
            pltpu.VMEM((2, _B1, _C), F32),
            pltpu.VMEM((2, _B1, 16), F32),
            pltpu.VMEM((2, _B1, _C), F32),
            pltpu.VMEM((16, _C), F32),
            pltpu.VMEM_SHARED((nrp, _C), F32),
            pltpu.SemaphoreType.DMA,
            pltpu.SemaphoreType.DMA,
            pltpu.SemaphoreType.DMA,
            pltpu.SemaphoreType.DMA,
        ],
    )
    def kern(src_hbm, dst_hbm, cb_hbm, ex_hbm, dp_hbm,
             srcall, dstall, rowv, colv, sbuf, dbuf, exbuf, pay, zbuf,
             dacc, semg0, semg1, semo0, semo1):
        cid = lax.axis_index("c")
        sid = lax.axis_index("s")
        wid = sid * 2 + cid
        base = wid * ew
        semg = (semg0, semg1)
        semo = (semo0, semo1)

        zv = jnp.zeros((16,), F32)
        ziv = jnp.zeros((16,), I32)

        pltpu.sync_copy(src_hbm.at[pl.ds(base, ew)], srcall)
        pltpu.sync_copy(dst_hbm.at[pl.ds(base, ew)], dstall)

        def zrow(i, carry):
            for k in range(_C // 16):
                zbuf[i, pl.ds(k * 16, 16)] = zv
            return carry

        lax.fori_loop(0, 16, zrow, 0)
        for q in range(rstride // 16):
            pltpu.sync_copy(zbuf, dacc.at[pl.ds(sid * rstride + q * 16, 16)])

        def zpay(i, carry):
            for s in range(2):
                for k in range(_C // 16):
                    pay[s, i, pl.ds(k * 16, 16)] = zv
            return carry

        lax.fori_loop(0, _B1, zpay, 0)
        for s in range(2):
            for g in range(_B1 // 16):
                colv[s, pl.ds(g * 16, 16)] = ziv
        plsc.subcore_barrier()

        def issue(c, s):
            cc = jnp.minimum(c, nch - 1)
            loff = cc * _B1
            pltpu.async_copy(cb_hbm.at[srcall.at[pl.ds(loff, _B1)]],
                             sbuf.at[s], semg[s])
            pltpu.async_copy(cb_hbm.at[dstall.at[pl.ds(loff, _B1)]],
                             dbuf.at[s], semg[s])

        issue(0, 0)
        issue(1, 1)

        def compute(c, s):
            for g in range(_B1 // 16):
                dv = dstall[pl.ds(c * _B1 + g * 16, 16)]
                rowv[s, pl.ds(g * 16, 16)] = lax.shift_right_logical(dv, 3)
                cv = lax.shift_left(jnp.bitwise_and(dv, 7), 4)
                colv[s, pl.ds(g * 16, 16)] = cv
                for l in range(16):
                    j = g * 16 + l
                    a = sbuf[s, j, pl.ds(0, 16)] + dbuf[s, j, pl.ds(16, 16)]
                    a = jnp.maximum(a, 0.2 * a)
                    ex = jnp.exp(a)
                    exbuf[s, j, :] = ex
                    pay[s, j, pl.ds(cv[l], 16)] = ex
            off = base + c * _B1
            pltpu.sync_copy(exbuf.at[s], ex_hbm.at[pl.ds(off, _B1)])
            pltpu.async_copy(pay.at[s], dacc.at[rowv.at[s]], semo[s],
                             add=True)
            issue(c + 2, s)

        def wait_gathers(s):
            pltpu.make_async_copy(
                cb_hbm.at[srcall.at[pl.ds(0, _B1)]],
                sbuf.at[s], semg[s]).wait()
            pltpu.make_async_copy(
                cb_hbm.at[dstall.at[pl.ds(0, _B1)]],
                dbuf.at[s], semg[s]).wait()

        def wait_outputs(s):
            pltpu.make_async_copy(
                pay.at[s], dacc.at[rowv.at[s]], semo[s]).wait()

        def zero_windows(s):
            for g in range(_B1 // 16):
                cvz = colv[s, pl.ds(g * 16, 16)]
                for l in range(16):
                    pay[s, g * 16 + l, pl.ds(cvz[l], 16)] = zv

        wait_gathers(0)
        compute(0, 0)
        wait_gathers(1)
        compute(1, 1)

        def slotstep(c, s):
            wait_gathers(s)
            wait_outputs(s)
            zero_windows(s)
            compute(c, s)

        def outer(i, carry):
            slotstep(2 * i, 0)
            slotstep(2 * i + 1, 1)
            return carry

        lax.fori_loop(1, nch // 2, outer, 0)
        slotstep(nch - 1, 0)
        for s in range(2):
            wait_gathers(s)
            wait_outputs(s)
        plsc.subcore_barrier()
        pltpu.sync_copy(dacc.at[pl.ds(sid * rstride, rstride)],
                        dp_hbm.at[cid, pl.ds(sid * rstride, rstride)])

    return kern(src, dst, combo)


def _sc_pass2(src, dst, ex_all, rd_pad, h, n):
    e = src.shape[0]
    b2 = _B2
    ew = e // _NW
    nch = ew // b2
    rstride = ((n // 16) // 8) * 8
    rsize = n - 15 * rstride
    hw = h.shape[1]
    mesh = plsc.VectorSubcoreMesh(core_axis_name="c", subcore_axis_name="s")

    @functools.partial(
        pl.kernel,
        out_type=jax.ShapeDtypeStruct((2, n, _C), F32),
        mesh=mesh,
        scratch_types=[
            pltpu.VMEM((2, b2), I32),
            pltpu.VMEM((2, b2), I32),
            pltpu.VMEM((2, b2), I32),
            pltpu.VMEM((b2, 16), F32),
            pltpu.VMEM((b2, 16), F32),
            pltpu.VMEM((b2, _C), F32),
            pltpu.VMEM((b2, _C), F32),
            pltpu.VMEM((b2, hw), F32),
            pltpu.VMEM((b2, hw), F32),
            pltpu.VMEM((b2, _C), F32),
            pltpu.VMEM((b2, _C), F32),
            pltpu.VMEM((16, _C), F32),
            pltpu.VMEM_SHARED((n, _C), F32),
            pltpu.SemaphoreType.DMA,
            pltpu.SemaphoreType.DMA,
            pltpu.SemaphoreType.DMA,
            pltpu.SemaphoreType.DMA,
            pltpu.SemaphoreType.DMA,
        ],
    )
    def kern(src_hbm, dst_hbm, ex_hbm, rd_hbm, h_hbm, out_hbm,
             srcvs, dstvs, dstw, exv0, exv1, rdv0, rdv1, hbuf0, hbuf1,
             mbuf0, mbuf1, zbuf, oacc,
             semg0, semg1, sems0, sems1, semi):
        cid = lax.axis_index("c")
        sid = lax.axis_index("s")
        wid = sid * 2 + cid
        base = wid * ew
        slots = (
            (exv0, rdv0, hbuf0, mbuf0, semg0, sems0),
            (exv1, rdv1, hbuf1, mbuf1, semg1, sems1),
        )

        zv = jnp.zeros((16,), F32)

        def zrow(i, carry):
            for k in range(_C // 16):
                zbuf[i, pl.ds(k * 16, 16)] = zv
            return carry

        lax.fori_loop(0, 16, zrow, 0)
        for q in range(rsize // 16):
            pltpu.sync_copy(zbuf, oacc.at[pl.ds(sid * rstride + q * 16, 16)])

        def zmb(i, carry):
            for k in range(_C // 16):
                mbuf0[i, pl.ds(k * 16, 16)] = zv
                mbuf1[i, pl.ds(k * 16, 16)] = zv
            return carry

        ziv = jnp.zeros((16,), I32)
        for s in range(2):
            for g in range(b2 // 16):
                srcvs[s, pl.ds(g * 16, 16)] = ziv
                dstvs[s, pl.ds(g * 16, 16)] = ziv
                dstw[s, pl.ds(g * 16, 16)] = ziv
        lax.fori_loop(0, b2, zmb, 0)
        plsc.subcore_barrier()

        def issue(c, s):
            exv, rdv, hbuf, _, semg, _ = slots[s]
            cc = jnp.minimum(c, nch - 1)
            off = base + cc * b2
            cpa = pltpu.async_copy(
                src_hbm.at[pl.ds(off, b2)], srcvs.at[s], semi)
            cpb = pltpu.async_copy(
                dst_hbm.at[pl.ds(off, b2)], dstvs.at[s], semi)
            cpa.wait()
            cpb.wait()
            pltpu.async_copy(ex_hbm.at[pl.ds(off, b2)], exv, semg)
            pltpu.async_copy(rd_hbm.at[dstvs.at[s]], rdv, semg)
            pltpu.async_copy(h_hbm.at[srcvs.at[s]], hbuf, semg)

        pltpu.async_copy(mbuf0, oacc.at[dstw.at[0]], sems0, add=True)
        pltpu.async_copy(mbuf1, oacc.at[dstw.at[1]], sems1, add=True)
        issue(0, 0)
        issue(1, 1)

        def slotstep(c, s):
            exv, rdv, hbuf, mbuf, semg, sems = slots[s]
            pltpu.make_async_copy(
                ex_hbm.at[pl.ds(base, b2)], exv, semg).wait()
            pltpu.make_async_copy(rd_hbm.at[dstvs.at[s]], rdv, semg).wait()
            pltpu.make_async_copy(h_hbm.at[srcvs.at[s]], hbuf, semg).wait()
            pltpu.make_async_copy(mbuf, oacc.at[dstw.at[s]], sems).wait()
            for g in range(b2 // 16):
                dstw[s, pl.ds(g * 16, 16)] = dstvs[s, pl.ds(g * 16, 16)]

            def edge(j, carry):
                wv = exv[j] * rdv[j, pl.ds(0, 16)]
                acc = [zv] * (_C // 16)
                for hh in range(_H):
                    sc = wv[hh]
                    for k in range(_C // 16):
                        acc[k] = acc[k] + sc * hbuf[j, pl.ds(hh * _C + k * 16, 16)]
                for k in range(_C // 16):
                    mbuf[j, pl.ds(k * 16, 16)] = acc[k]
                return carry

            lax.fori_loop(0, b2, edge, 0, unroll=2)
            pltpu.async_copy(mbuf, oacc.at[dstw.at[s]], sems, add=True)
            issue(c + 2, s)

        def outer(i, carry):
            slotstep(2 * i, 0)
            slotstep(2 * i + 1, 1)
            return carry

        lax.fori_loop(0, nch // 2, outer, 0)
        if nch % 2 == 1:
            slotstep(nch - 1, 0)
        for s in range(2):
            exv, rdv, hbuf, mbuf, semg, sems = slots[s]
            pltpu.make_async_copy(
                ex_hbm.at[pl.ds(base, b2)], exv, semg).wait()
            pltpu.make_async_copy(rd_hbm.at[dstvs.at[s]], rdv, semg).wait()
            pltpu.make_async_copy(h_hbm.at[srcvs.at[s]], hbuf, semg).wait()
            pltpu.make_async_copy(mbuf, oacc.at[dstw.at[s]], sems).wait()
        plsc.subcore_barrier()
        pltpu.sync_copy(oacc.at[pl.ds(sid * rstride, rsize)],
                        out_hbm.at[cid, pl.ds(sid * rstride, rsize)])

    return kern(src, dst, ex_all, rd_pad, h)


def kernel(x, edge_index, edge_attr, edge_type, rel_embed, num_nodes,
           Wn, bn, We, be, Wg, att_src, att_dst, bg, Wr, br):
    n, d = x.shape

    Wg3 = Wg.reshape(d, _H, _C)
    Vs = jnp.einsum('dhk,hk->dh', Wg3, att_src)
    Vd = jnp.einsum('dhk,hk->dh', Wg3, att_dst)
    Vc = jnp.concatenate(
        [Vs, Vs, Vd, Vd, jnp.zeros((d, _C - 32), F32)], axis=1)
    bn2 = bn.reshape(1, -1)
    bg2 = bg.reshape(1, -1)
    br2 = br.reshape(1, -1)

    src = edge_index[0]
    dst = edge_index[1]

    h, combo = _tc_stage1(x, Wn, bn2, Wg, Vc)
    ex_all, dpart = _sc_pass1(src, dst, combo, n)
    dpart = dpart[:, :n // 8, :].reshape(2, n, 16)
    rd_pad, wl_dup = _tc_stage2(combo, dpart)
    opart = _sc_pass2(src, dst, ex_all, rd_pad, h, n)
    x1 = _tc_stage3(opart, h, wl_dup, bg2)
    rel_out = _tc_rel(rel_embed, Wr, br2)
    return (x1, rel_out)


# speedup vs baseline: 1.4970x; 1.0006x over previous
import functools

import jax
import jax.numpy as jnp
from jax import lax
from jax.experimental import pallas as pl
from jax.experimental.pallas import tpu as pltpu
from jax.experimental.pallas import tpu_sc as plsc

F32 = jnp.float32
I32 = jnp.int32

_H = 8
_C = 128
_NW = 32
_B1 = 80
_B2 = 16


def _tc_stage1(x, Wn, bn2, Wg, Vc):
    n, d = x.shape
    bn_rows = 2000
    grid = (n // bn_rows,)
    hw = Wg.shape[1]

    def body(x_ref, wn_ref, bn_ref, wg_ref, vc_ref, h_ref, cb_ref):
        h0 = jnp.dot(x_ref[...], wn_ref[...],
                     preferred_element_type=F32) + bn_ref[...]
        h_ref[...] = jnp.dot(h0, wg_ref[...], preferred_element_type=F32)
        cb_ref[...] = jnp.dot(h0, vc_ref[...], preferred_element_type=F32)

    return pl.pallas_call(
        body,
        grid=grid,
        in_specs=[
            pl.BlockSpec((bn_rows, d), lambda i: (i, 0)),
            pl.BlockSpec((d, d), lambda i: (0, 0)),
            pl.BlockSpec((1, d), lambda i: (0, 0)),
            pl.BlockSpec((d, hw), lambda i: (0, 0)),
            pl.BlockSpec((d, _C), lambda i: (0, 0)),
        ],
        out_specs=[
            pl.BlockSpec((bn_rows, hw), lambda i: (i, 0)),
            pl.BlockSpec((bn_rows, _C), lambda i: (i, 0)),
        ],
        out_shape=[
            jax.ShapeDtypeStruct((n, hw), F32),
            jax.ShapeDtypeStruct((n, _C), F32),
        ],
    )(x, Wn, bn2, Wg, Vc)


def _tc_stage2(combo, dpart):
    n = combo.shape[0]
    bn_rows = 2000
    grid = (n // bn_rows,)

    def body(cb_ref, dp_ref, rd_ref, wl_ref):
        cb = cb_ref[...]
        al = cb[:, 0:16] + cb[:, 16:32]
        exl = jnp.exp(jnp.maximum(al, 0.2 * al))
        den = dp_ref[0][:, 0:16] + dp_ref[1][:, 0:16] + exl
        rd = 1.0 / den
        rd_ref[...] = jnp.concatenate(
            [rd, jnp.zeros((rd.shape[0], _C - 16), F32)], axis=1)
        wl_ref[...] = exl * rd

    return pl.pallas_call(
        body,
        grid=grid,
        in_specs=[
            pl.BlockSpec((bn_rows, _C), lambda i: (i, 0)),
            pl.BlockSpec((2, bn_rows, _C), lambda i: (0, i, 0)),
        ],
        out_specs=[
            pl.BlockSpec((bn_rows, _C), lambda i: (i, 0)),
            pl.BlockSpec((bn_rows, 16), lambda i: (i, 0)),
        ],
        out_shape=[
            jax.ShapeDtypeStruct((n, _C), F32),
            jax.ShapeDtypeStruct((n, 16), F32),
        ],
    )(combo, dpart)


def _tc_stage3(opart, h, wl_dup, bg2):
    n = h.shape[0]
    bn_rows = 2000
    grid = (n // bn_rows,)
    hw = h.shape[1]

    def body(op_ref, h_ref, wl_ref, bg_ref, o_ref):
        acc = op_ref[0] + op_ref[1]
        wl = wl_ref[...]
        hv = h_ref[...]
        for hh in range(_H):
            acc = acc + wl[:, hh:hh + 1] * hv[:, hh * _C:(hh + 1) * _C]
        o_ref[...] = jnp.tanh(acc * (1.0 / _H) + bg_ref[...])

    return pl.pallas_call(
        body,
        grid=grid,
        in_specs=[
            pl.BlockSpec((2, bn_rows, _C), lambda i: (0, i, 0)),
            pl.BlockSpec((bn_rows, hw), lambda i: (i, 0)),
            pl.BlockSpec((bn_rows, 16), lambda i: (i, 0)),
            pl.BlockSpec((1, _C), lambda i: (0, 0)),
        ],
        out_specs=pl.BlockSpec((bn_rows, _C), lambda i: (i, 0)),
        out_shape=jax.ShapeDtypeStruct((n, _C), F32),
    )(opart, h, wl_dup, bg2)


def _tc_rel(rel_embed, Wr, br2):
    r, d = rel_embed.shape

    def body(re_ref, wr_ref, br_ref, o_ref):
        o_ref[...] = jnp.dot(re_ref[...], wr_ref[...],
                             preferred_element_type=F32) + br_ref[...]

    return pl.pallas_call(
        body,
        out_shape=jax.ShapeDtypeStruct((r, d), F32),
    )(rel_embed, Wr, br2)


def _sc_pass1(src, dst, combo, n):
    e = src.shape[0]
    ew = e // _NW
    nr = n // 8
    nrp = ((nr + 127) // 128) * 128
    rstride = nrp // 16
    nch = ew // _B1
    mesh = plsc.VectorSubcoreMesh(core_axis_name="c", subcore_axis_name="s")

    @functools.partial(
        pl.kernel,
        out_type=(
            jax.ShapeDtypeStruct((e, 16), F32),
            jax.ShapeDtypeStruct((2, nrp, _C), F32),
        ),
        mesh=mesh,
        scratch_types=[
            pltpu.VMEM((ew,), I32),
            pltpu.VMEM((ew,), I32),
            pltpu.VMEM((2, _B1), I32),
            pltpu.VMEM((2, _B1), I32),
            pltpu.VMEM((2, _B1, _C), F32),
            pltpu.VMEM((2, _B1, _C), F32),
            pltpu.VMEM((2, _B1, 16), F32),
            pltpu.VMEM((2, _B1, _C), F32),
            pltpu.VMEM((16, _C), F32),
            pltpu.VMEM_SHARED((nrp, _C), F32),
            pltpu.SemaphoreType.DMA,
            pltpu.SemaphoreType.DMA,
            pltpu.SemaphoreType.DMA,
            pltpu.SemaphoreType.DMA,
        ],
    )
    def kern(src_hbm, dst_hbm, cb_hbm, ex_hbm, dp_hbm,
             srcall, dstall, rowv, colv, sbuf, dbuf, exbuf, pay, zbuf,
             dacc, semg0, semg1, semo0, semo1):
        cid = lax.axis_index("c")
        sid = lax.axis_index("s")
        wid = sid * 2 + cid
        base = wid * ew
        semg = (semg0, semg1)
        semo = (semo0, semo1)

        zv = jnp.zeros((16,), F32)
        ziv = jnp.zeros((16,), I32)

        pltpu.sync_copy(src_hbm.at[pl.ds(base, ew)], srcall)
        pltpu.sync_copy(dst_hbm.at[pl.ds(base, ew)], dstall)

        def zrow(i, carry):
            for k in range(_C // 16):
                zbuf[i, pl.ds(k * 16, 16)] = zv
            return carry

        lax.fori_loop(0, 16, zrow, 0)
        for q in range(rstride // 16):
            pltpu.sync_copy(zbuf, dacc.at[pl.ds(sid * rstride + q * 16, 16)])

        def zpay(i, carry):
            for s in range(2):
                for k in range(_C // 16):
                    pay[s, i, pl.ds(k * 16, 16)] = zv
            return carry

        lax.fori_loop(0, _B1, zpay, 0)
        for s in range(2):
            for g in range(_B1 // 16):
                colv[s, pl.ds(g * 16, 16)] = ziv
        plsc.subcore_barrier()

        def issue(c, s):
            cc = jnp.minimum(c, nch - 1)
            loff = cc * _B1
            pltpu.async_copy(cb_hbm.at[srcall.at[pl.ds(loff, _B1)]],
                             sbuf.at[s], semg[s])
            pltpu.async_copy(cb_hbm.at[dstall.at[pl.ds(loff, _B1)]],
                             dbuf.at[s], semg[s])

        issue(0, 0)
        issue(1, 1)

        def compute(c, s):
            for g in range(_B1 // 16):
                dv = dstall[pl.ds(c * _B1 + g * 16, 16)]
                rowv[s, pl.ds(g * 16, 16)] = lax.shift_right_logical(dv, 3)
                cv = lax.shift_left(jnp.bitwise_and(dv, 7), 4)
                colv[s, pl.ds(g * 16, 16)] = cv
                for l in range(16):
                    j = g * 16 + l
                    a = sbuf[s, j, pl.ds(0, 16)] + dbuf[s, j, pl.ds(16, 16)]
                    a = jnp.maximum(a, 0.2 * a)
                    ex = jnp.exp(a)
                    exbuf[s, j, :] = ex
                    pay[s, j, pl.ds(cv[l], 16)] = ex
            off = base + c * _B1
            pltpu.sync_copy(exbuf.at[s], ex_hbm.at[pl.ds(off, _B1)])
            pltpu.async_copy(pay.at[s], dacc.at[rowv.at[s]], semo[s],
                             add=True)
            issue(c + 2, s)

        def wait_gathers(s):
            pltpu.make_async_copy(
                cb_hbm.at[srcall.at[pl.ds(0, _B1)]],
                sbuf.at[s], semg[s]).wait()
            pltpu.make_async_copy(
                cb_hbm.at[dstall.at[pl.ds(0, _B1)]],
                dbuf.at[s], semg[s]).wait()

        def wait_outputs(s):
            pltpu.make_async_copy(
                pay.at[s], dacc.at[rowv.at[s]], semo[s]).wait()

        def zero_windows(s):
            for g in range(_B1 // 16):
                cvz = colv[s, pl.ds(g * 16, 16)]
                for l in range(16):
                    pay[s, g * 16 + l, pl.ds(cvz[l], 16)] = zv

        wait_gathers(0)
        compute(0, 0)
        wait_gathers(1)
        compute(1, 1)

        def slotstep(c, s):
            wait_gathers(s)
            wait_outputs(s)
            zero_windows(s)
            compute(c, s)

        def outer(i, carry):
            slotstep(2 * i, 0)
            slotstep(2 * i + 1, 1)
            return carry

        lax.fori_loop(1, nch // 2, outer, 0)
        slotstep(nch - 1, 0)
        for s in range(2):
            wait_gathers(s)
            wait_outputs(s)
        plsc.subcore_barrier()
        pltpu.sync_copy(dacc.at[pl.ds(sid * rstride, rstride)],
                        dp_hbm.at[cid, pl.ds(sid * rstride, rstride)])

    return kern(src, dst, combo)


def _sc_pass2(src, dst, ex_all, rd_pad, h, n):
    e = src.shape[0]
    b2 = _B2
    ew = e // _NW
    nch = ew // b2
    rstride = ((n // 16) // 8) * 8
    rsize = n - 15 * rstride
    hw = h.shape[1]
    mesh = plsc.VectorSubcoreMesh(core_axis_name="c", subcore_axis_name="s")

    @functools.partial(
        pl.kernel,
        out_type=jax.ShapeDtypeStruct((2, n, _C), F32),
        mesh=mesh,
        scratch_types=[
            pltpu.VMEM((2, b2), I32),
            pltpu.VMEM((2, b2), I32),
            pltpu.VMEM((2, b2), I32),
            pltpu.VMEM((b2, 16), F32),
            pltpu.VMEM((b2, 16), F32),
            pltpu.VMEM((b2, _C), F32),
            pltpu.VMEM((b2, _C), F32),
            pltpu.VMEM((b2, hw), F32),
            pltpu.VMEM((b2, hw), F32),
            pltpu.VMEM((b2, _C), F32),
            pltpu.VMEM((b2, _C), F32),
            pltpu.VMEM((16, _C), F32),
            pltpu.VMEM_SHARED((n, _C), F32),
            pltpu.SemaphoreType.DMA,
            pltpu.SemaphoreType.DMA,
            pltpu.SemaphoreType.DMA,
            pltpu.SemaphoreType.DMA,
            pltpu.SemaphoreType.DMA,
        ],
    )
    def kern(src_hbm, dst_hbm, ex_hbm, rd_hbm, h_hbm, out_hbm,
             srcvs, dstvs, dstw, exv0, exv1, rdv0, rdv1, hbuf0, hbuf1,
             mbuf0, mbuf1, zbuf, oacc,
             semg0, semg1, sems0, sems1, semi):
        cid = lax.axis_index("c")
        sid = lax.axis_index("s")
        wid = sid * 2 + cid
        base = wid * ew
        slots = (
            (exv0, rdv0, hbuf0, mbuf0, semg0, sems0),
            (exv1, rdv1, hbuf1, mbuf1, semg1, sems1),
        )

        zv = jnp.zeros((16,), F32)

        def zrow(i, carry):
            for k in range(_C // 16):
                zbuf[i, pl.ds(k * 16, 16)] = zv
            return carry

        lax.fori_loop(0, 16, zrow, 0)
        for q in range(rsize // 16):
            pltpu.sync_copy(zbuf, oacc.at[pl.ds(sid * rstride + q * 16, 16)])

        def zmb(i, carry):
            for k in range(_C // 16):
                mbuf0[i, pl.ds(k * 16, 16)] = zv
                mbuf1[i, pl.ds(k * 16, 16)] = zv
            return carry

        ziv = jnp.zeros((16,), I32)
        for s in range(2):
            for g in range(b2 // 16):
                srcvs[s, pl.ds(g * 16, 16)] = ziv
                dstvs[s, pl.ds(g * 16, 16)] = ziv
                dstw[s, pl.ds(g * 16, 16)] = ziv
        lax.fori_loop(0, b2, zmb, 0)
        plsc.subcore_barrier()

        def issue(c, s):
            exv, rdv, hbuf, _, semg, _ = slots[s]
            cc = jnp.minimum(c, nch - 1)
            off = base + cc * b2
            cpa = pltpu.async_copy(
                src_hbm.at[pl.ds(off, b2)], srcvs.at[s], semi)
            cpb = pltpu.async_copy(
                dst_hbm.at[pl.ds(off, b2)], dstvs.at[s], semi)
            cpa.wait()
            cpb.wait()
            pltpu.async_copy(ex_hbm.at[pl.ds(off, b2)], exv, semg)
            pltpu.async_copy(rd_hbm.at[dstvs.at[s]], rdv, semg)
            pltpu.async_copy(h_hbm.at[srcvs.at[s]], hbuf, semg)

        pltpu.async_copy(mbuf0, oacc.at[dstw.at[0]], sems0, add=True)
        pltpu.async_copy(mbuf1, oacc.at[dstw.at[1]], sems1, add=True)
        issue(0, 0)
        issue(1, 1)

        def slotstep(c, s):
            exv, rdv, hbuf, mbuf, semg, sems = slots[s]
            pltpu.make_async_copy(
                ex_hbm.at[pl.ds(base, b2)], exv, semg).wait()
            pltpu.make_async_copy(rd_hbm.at[dstvs.at[s]], rdv, semg).wait()
            pltpu.make_async_copy(h_hbm.at[srcvs.at[s]], hbuf, semg).wait()
            pltpu.make_async_copy(mbuf, oacc.at[dstw.at[s]], sems).wait()
            for g in range(b2 // 16):
                dstw[s, pl.ds(g * 16, 16)] = dstvs[s, pl.ds(g * 16, 16)]

            def edge(j, carry):
                wv = exv[j] * rdv[j, pl.ds(0, 16)]
                acc = [zv] * (_C // 16)
                for hh in range(_H):
                    sc = wv[hh]
                    for k in range(_C // 16):
                        acc[k] = acc[k] + sc * hbuf[j, pl.ds(hh * _C + k * 16, 16)]
                for k in range(_C // 16):
                    mbuf[j, pl.ds(k * 16, 16)] = acc[k]
                return carry

            lax.fori_loop(0, b2, edge, 0, unroll=4)
            pltpu.async_copy(mbuf, oacc.at[dstw.at[s]], sems, add=True)
            issue(c + 2, s)

        def outer(i, carry):
            slotstep(2 * i, 0)
            slotstep(2 * i + 1, 1)
            return carry

        lax.fori_loop(0, nch // 2, outer, 0)
        if nch % 2 == 1:
            slotstep(nch - 1, 0)
        for s in range(2):
            exv, rdv, hbuf, mbuf, semg, sems = slots[s]
            pltpu.make_async_copy(
                ex_hbm.at[pl.ds(base, b2)], exv, semg).wait()
            pltpu.make_async_copy(rd_hbm.at[dstvs.at[s]], rdv, semg).wait()
            pltpu.make_async_copy(h_hbm.at[srcvs.at[s]], hbuf, semg).wait()
            pltpu.make_async_copy(mbuf, oacc.at[dstw.at[s]], sems).wait()
        plsc.subcore_barrier()
        pltpu.sync_copy(oacc.at[pl.ds(sid * rstride, rsize)],
                        out_hbm.at[cid, pl.ds(sid * rstride, rsize)])

    return kern(src, dst, ex_all, rd_pad, h)


def kernel(x, edge_index, edge_attr, edge_type, rel_embed, num_nodes,
           Wn, bn, We, be, Wg, att_src, att_dst, bg, Wr, br):
    n, d = x.shape

    Wg3 = Wg.reshape(d, _H, _C)
    Vs = jnp.einsum('dhk,hk->dh', Wg3, att_src)
    Vd = jnp.einsum('dhk,hk->dh', Wg3, att_dst)
    Vc = jnp.concatenate(
        [Vs, Vs, Vd, Vd, jnp.zeros((d, _C - 32), F32)], axis=1)
    bn2 = bn.reshape(1, -1)
    bg2 = bg.reshape(1, -1)
    br2 = br.reshape(1, -1)

    src = edge_index[0]
    dst = edge_index[1]

    h, combo = _tc_stage1(x, Wn, bn2, Wg, Vc)
    ex_all, dpart = _sc_pass1(src, dst, combo, n)
    dpart = dpart[:, :n // 8, :].reshape(2, n, 16)
    rd_pad, wl_dup = _tc_stage2(combo, dpart)
    opart = _sc_pass2(src, dst, ex_all, rd_pad, h, n)
    x1 = _tc_stage3(opart, h, wl_dup, bg2)
    rel_out = _tc_rel(rel_embed, Wr, br2)
    return (x1, rel_out)
